# baseline (speedup 1.0000x reference)
import functools

import jax
import jax.numpy as jnp
from jax import lax
from jax.experimental import pallas as pl
from jax.experimental.pallas import tpu as pltpu
from jax.experimental.pallas import tpu_sc as plsc

N_FIELDS = 26
VOCAB = 100000
EMBED_D = 32
BATCH = 16384
TAB_ROWS = N_FIELDS * VOCAB
NUM_CORES = 2
NUM_SUBCORES = 16
NUM_WORKERS = NUM_CORES * NUM_SUBCORES
LANES = 16

GROUP = 4
TAB_GROUPS = TAB_ROWS // GROUP
GROUP_W = GROUP * EMBED_D

TBLK = 512
NBLK = 5078
TAIL = TAB_ROWS - NBLK * TBLK

ROWS_W = BATCH // NUM_WORKERS
ELEMS_W = ROWS_W * N_FIELDS
GW = 4 * N_FIELDS
NG = ROWS_W // 4
PAIR = 2 * GW
NBUF = 2


def _tree_sum(vals):
    while len(vals) > 1:
        nxt = [vals[i] + vals[i + 1] for i in range(0, len(vals) - 1, 2)]
        if len(vals) % 2:
            nxt.append(vals[-1])
        vals = nxt
    return vals[0]


@functools.partial(
    pl.kernel,
    out_type=jax.ShapeDtypeStruct((TAB_GROUPS, GROUP_W), jnp.float32),
    mesh=plsc.VectorSubcoreMesh(core_axis_name="c", subcore_axis_name="s"),
    compiler_params=pltpu.CompilerParams(needs_layout_passes=False),
    scratch_types=[
        pltpu.VMEM((GROUP, 8, TBLK), jnp.float32),
        pltpu.VMEM((TBLK // GROUP, GROUP_W), jnp.float32),
        pltpu.VMEM((TAIL // GROUP, GROUP_W), jnp.float32),
        pltpu.SemaphoreType.DMA,
        pltpu.SemaphoreType.DMA,
    ],
)
def _stage(tab_t_hbm, tail_hbm, out_hbm, in_v, out_t, tail_v, sem, sem2):
    wid = lax.axis_index("s") * NUM_CORES + lax.axis_index("c")
    iota = lax.iota(jnp.int32, LANES)
    iv_row = lax.shift_right_logical(iota, 2)
    iv_col = (iota & 3) * EMBED_D

    @pl.when(wid == 0)
    def _():
        pltpu.sync_copy(tail_hbm, tail_v)
        pltpu.sync_copy(tail_v, out_hbm.at[pl.ds(NBLK * TBLK // GROUP,
                                                 TAIL // GROUP), :])

    def block(i, carry):
        j = i * NUM_WORKERS + wid

        @pl.when(j < NBLK)
        def _():
            c0 = j * TBLK
            for db in range(GROUP):
                pltpu.async_copy(
                    tab_t_hbm.at[pl.ds(db * 8, 8), pl.ds(c0, TBLK)],
                    in_v.at[db], sem)
            for db in range(GROUP):
                pltpu.make_async_copy(
                    tab_t_hbm.at[pl.ds(db * 8, 8), pl.ds(c0, TBLK)],
                    in_v.at[db], sem).wait()
            for db in range(GROUP):
                for d in range(8):
                    dim = db * 8 + d
                    col = iv_col + dim
                    for rs in range(TBLK // LANES):
                        plsc.store_scatter(
                            out_t, [rs * GROUP + iv_row, col],
                            in_v[db, d, pl.ds(rs * LANES, LANES)])
            pltpu.sync_copy(
                out_t, out_hbm.at[pl.ds(j * (TBLK // GROUP),
                                        TBLK // GROUP), :])
        return carry

    lax.fori_loop(0, (NBLK + NUM_WORKERS - 1) // NUM_WORKERS, block, 0)


def _gather_body(inp_hbm, tab_hbm, out_hbm, idx_v, gid_v, pat_v, acc_v,
                 buf_v, sem_in, sem0, sem1):
    wid = lax.axis_index("s") * NUM_CORES + lax.axis_index("c")
    sems = (sem0, sem1)

    in_cp = pltpu.async_copy(
        inp_hbm.at[pl.ds(wid * ELEMS_W, ELEMS_W)],
        idx_v.at[pl.ds(0, ELEMS_W)], sem_in)

    iota = lax.iota(jnp.int32, LANES)
    for m in range(PAIR // LANES):
        pat_v[m, :] = ((m * LANES + iota) % N_FIELDS) * VOCAB
    in_cp.wait()

    def adjust(p):
        for m in range(PAIR // LANES):
            sl = pl.ds(p * PAIR + m * LANES, LANES)
            v = idx_v[sl] + pat_v[m, :]
            idx_v[sl] = v
            gid_v[sl] = lax.shift_right_logical(v, 2)

    def start(k, b):
        pltpu.async_copy(
            tab_hbm.at[gid_v.at[pl.ds(k * GW, GW)]], buf_v.at[b], sems[b])

    adjust(0)
    start(0, 0)
    start(1, 1)

    def ring(g, carry):
        for b in range(NBUF):
            k = NBUF * g + b
            pltpu.make_async_copy(
                tab_hbm.at[gid_v.at[pl.ds(k * GW, GW)]], buf_v.at[b],
                sems[b]).wait()
            if b == 0:
                @pl.when(g + 1 < NG // NBUF)
                def _():
                    adjust(g + 1)
            vs = [idx_v[pl.ds(k * GW + j * LANES, LANES)]
                  for j in range(7)]
            for br in range(4):
                arow = 4 * k + br
                cols = [(vs[(br * N_FIELDS + f) // LANES]
                         [(br * N_FIELDS + f) % LANES] & 3) * EMBED_D
                        for f in range(N_FIELDS)]
                for h in range(EMBED_D // LANES):
                    acc_v[pl.ds(arow * EMBED_D + h * LANES, LANES)] = (
                        _tree_sum(
                            [buf_v[b, br * N_FIELDS + f,
                                   pl.ds(cols[f] + h * LANES, LANES)]
                             for f in range(N_FIELDS)]))

            @pl.when(k + NBUF < NG)
            def _():
                start(k + NBUF, b)
        return carry

    lax.fori_loop(0, NG // NBUF, ring, 0)
    pltpu.sync_copy(acc_v, out_hbm.at[pl.ds(wid * ROWS_W * EMBED_D,
                                            ROWS_W * EMBED_D)])


@functools.partial(
    pl.kernel,
    out_type=jax.ShapeDtypeStruct((BATCH * EMBED_D,), jnp.float32),
    mesh=plsc.VectorSubcoreMesh(core_axis_name="c", subcore_axis_name="s"),
    scratch_types=[
        pltpu.VMEM((ELEMS_W + LANES,), jnp.int32),
        pltpu.VMEM((ELEMS_W,), jnp.int32),
        pltpu.VMEM((PAIR // LANES, LANES), jnp.int32),
        pltpu.VMEM((ROWS_W * EMBED_D,), jnp.float32),
        pltpu.VMEM((NBUF, GW, GROUP_W), jnp.float32),
        pltpu.SemaphoreType.DMA,
        pltpu.SemaphoreType.DMA,
        pltpu.SemaphoreType.DMA,
    ],
)
def _attr_embed(inp_hbm, tab_hbm, out_hbm, idx_v, gid_v, pat_v, acc_v,
                buf_v, sem_in, sem0, sem1):
    _gather_body(inp_hbm, tab_hbm, out_hbm, idx_v, gid_v, pat_v, acc_v,
                 buf_v, sem_in, sem0, sem1)


def kernel(inputs, table):
    assert inputs.shape == (BATCH, N_FIELDS) and inputs.dtype == jnp.int32
    tab_t = table.T
    tail = table[NBLK * TBLK:, :].reshape(TAIL // GROUP, GROUP_W)
    staged = _stage(tab_t, tail)
    out = _attr_embed(inputs.reshape(-1), staged)
    return out.reshape(BATCH, EMBED_D)


# speedup vs baseline: 1.0742x; 1.0742x over previous
import functools

import jax
import jax.numpy as jnp
from jax import lax
from jax.experimental import pallas as pl
from jax.experimental.pallas import tpu as pltpu
from jax.experimental.pallas import tpu_sc as plsc

N_FIELDS = 26
VOCAB = 100000
EMBED_D = 32
BATCH = 16384
TAB_ROWS = N_FIELDS * VOCAB
NUM_CORES = 2
NUM_SUBCORES = 16
NUM_WORKERS = NUM_CORES * NUM_SUBCORES
LANES = 16

GROUP = 4
TAB_GROUPS = TAB_ROWS // GROUP
GROUP_W = GROUP * EMBED_D

TBLK = 1024
NBLK = 2539
TAIL = TAB_ROWS - NBLK * TBLK

ROWS_W = BATCH // NUM_WORKERS
ELEMS_W = ROWS_W * N_FIELDS
GW = 4 * N_FIELDS
NG = ROWS_W // 4
PAIR = 2 * GW
NBUF = 2


def _tree_sum(vals):
    while len(vals) > 1:
        nxt = [vals[i] + vals[i + 1] for i in range(0, len(vals) - 1, 2)]
        if len(vals) % 2:
            nxt.append(vals[-1])
        vals = nxt
    return vals[0]


@functools.partial(
    pl.kernel,
    out_type=jax.ShapeDtypeStruct((TAB_GROUPS, GROUP_W), jnp.float32),
    mesh=plsc.VectorSubcoreMesh(core_axis_name="c", subcore_axis_name="s"),
    compiler_params=pltpu.CompilerParams(needs_layout_passes=False),
    scratch_types=[
        pltpu.VMEM((2, EMBED_D, TBLK), jnp.float32),
        pltpu.VMEM((TBLK // GROUP, GROUP_W), jnp.float32),
        pltpu.VMEM((TAIL // GROUP, GROUP_W), jnp.float32),
        pltpu.SemaphoreType.DMA,
        pltpu.SemaphoreType.DMA,
        pltpu.SemaphoreType.DMA,
    ],
)
def _stage(tab_t_hbm, tail_hbm, out_hbm, in_v, out_t, tail_v,
           sem0, sem1, sem2):
    wid = lax.axis_index("s") * NUM_CORES + lax.axis_index("c")
    sems = (sem0, sem1)
    iota = lax.iota(jnp.int32, LANES)
    iv_row = lax.shift_right_logical(iota, 2)
    iv_col = (iota & 3) * EMBED_D

    @pl.when(wid == 0)
    def _():
        pltpu.sync_copy(tail_hbm, tail_v)
        pltpu.sync_copy(tail_v, out_hbm.at[pl.ds(NBLK * TBLK // GROUP,
                                                 TAIL // GROUP), :])

    nit = (NBLK + NUM_WORKERS - 1) // NUM_WORKERS

    def fetch(i, bslot):
        j = i * NUM_WORKERS + wid

        @pl.when(j < NBLK)
        def _():
            for d in range(EMBED_D):
                pltpu.async_copy(
                    tab_t_hbm.at[d, pl.ds(j * TBLK, TBLK)],
                    in_v.at[bslot, d], sems[bslot])

    def drain(i, bslot):
        j = i * NUM_WORKERS + wid

        @pl.when(j < NBLK)
        def _():
            for d in range(EMBED_D):
                pltpu.make_async_copy(
                    tab_t_hbm.at[d, pl.ds(j * TBLK, TBLK)],
                    in_v.at[bslot, d], sems[bslot]).wait()

    fetch(0, 0)

    def block(i, carry):
        for bslot in range(2):
            j = i * NUM_WORKERS * 2 + bslot * NUM_WORKERS + wid
            ii = 2 * i + bslot
            drain(ii, bslot)
            fetch(ii + 1, 1 - bslot)

            @pl.when(j < NBLK)
            def _():
                def rs_body(rs, c):
                    rows = rs * GROUP + iv_row
                    for d in range(EMBED_D):
                        plsc.store_scatter(
                            out_t, [rows, iv_col + d],
                            in_v[bslot, d, pl.ds(rs * LANES, LANES)])
                    return c
                lax.fori_loop(0, TBLK // LANES, rs_body, 0)
                pltpu.sync_copy(
                    out_t, out_hbm.at[pl.ds(j * (TBLK // GROUP),
                                            TBLK // GROUP), :])
        return carry

    lax.fori_loop(0, (nit + 1) // 2, block, 0)


def _gather_body(inp_hbm, tab_hbm, out_hbm, idx_v, gid_v, pat_v, acc_v,
                 buf_v, sem_in, sem0, sem1):
    wid = lax.axis_index("s") * NUM_CORES + lax.axis_index("c")
    sems = (sem0, sem1)

    in_cp = pltpu.async_copy(
        inp_hbm.at[pl.ds(wid * ELEMS_W, ELEMS_W)],
        idx_v.at[pl.ds(0, ELEMS_W)], sem_in)

    iota = lax.iota(jnp.int32, LANES)
    for m in range(PAIR // LANES):
        pat_v[m, :] = ((m * LANES + iota) % N_FIELDS) * VOCAB
    in_cp.wait()

    def adjust(p):
        for m in range(PAIR // LANES):
            sl = pl.ds(p * PAIR + m * LANES, LANES)
            v = idx_v[sl] + pat_v[m, :]
            idx_v[sl] = v
            gid_v[sl] = lax.shift_right_logical(v, 2)

    def start(k, b):
        pltpu.async_copy(
            tab_hbm.at[gid_v.at[pl.ds(k * GW, GW)]], buf_v.at[b], sems[b])

    adjust(0)
    start(0, 0)
    start(1, 1)

    def ring(g, carry):
        for b in range(NBUF):
            k = NBUF * g + b
            pltpu.make_async_copy(
                tab_hbm.at[gid_v.at[pl.ds(k * GW, GW)]], buf_v.at[b],
                sems[b]).wait()
            if b == 0:
                @pl.when(g + 1 < NG // NBUF)
                def _():
                    adjust(g + 1)
            vs = [idx_v[pl.ds(k * GW + j * LANES, LANES)]
                  for j in range(7)]
            for br in range(4):
                arow = 4 * k + br
                cols = [(vs[(br * N_FIELDS + f) // LANES]
                         [(br * N_FIELDS + f) % LANES] & 3) * EMBED_D
                        for f in range(N_FIELDS)]
                for h in range(EMBED_D // LANES):
                    acc_v[pl.ds(arow * EMBED_D + h * LANES, LANES)] = (
                        _tree_sum(
                            [buf_v[b, br * N_FIELDS + f,
                                   pl.ds(cols[f] + h * LANES, LANES)]
                             for f in range(N_FIELDS)]))

            @pl.when(k + NBUF < NG)
            def _():
                start(k + NBUF, b)
        return carry

    lax.fori_loop(0, NG // NBUF, ring, 0)
    pltpu.sync_copy(acc_v, out_hbm.at[pl.ds(wid * ROWS_W * EMBED_D,
                                            ROWS_W * EMBED_D)])


@functools.partial(
    pl.kernel,
    out_type=jax.ShapeDtypeStruct((BATCH * EMBED_D,), jnp.float32),
    mesh=plsc.VectorSubcoreMesh(core_axis_name="c", subcore_axis_name="s"),
    scratch_types=[
        pltpu.VMEM((ELEMS_W + LANES,), jnp.int32),
        pltpu.VMEM((ELEMS_W,), jnp.int32),
        pltpu.VMEM((PAIR // LANES, LANES), jnp.int32),
        pltpu.VMEM((ROWS_W * EMBED_D,), jnp.float32),
        pltpu.VMEM((NBUF, GW, GROUP_W), jnp.float32),
        pltpu.SemaphoreType.DMA,
        pltpu.SemaphoreType.DMA,
        pltpu.SemaphoreType.DMA,
    ],
)
def _attr_embed(inp_hbm, tab_hbm, out_hbm, idx_v, gid_v, pat_v, acc_v,
                buf_v, sem_in, sem0, sem1):
    _gather_body(inp_hbm, tab_hbm, out_hbm, idx_v, gid_v, pat_v, acc_v,
                 buf_v, sem_in, sem0, sem1)


def kernel(inputs, table):
    assert inputs.shape == (BATCH, N_FIELDS) and inputs.dtype == jnp.int32
    tab_t = table.T
    tail = table[NBLK * TBLK:, :].reshape(TAIL // GROUP, GROUP_W)
    staged = _stage(tab_t, tail)
    out = _attr_embed(inputs.reshape(-1), staged)
    return out.reshape(BATCH, EMBED_D)


# speedup vs baseline: 1.1273x; 1.0494x over previous
import functools

import jax
import jax.numpy as jnp
from jax import lax
from jax.experimental import pallas as pl
from jax.experimental.pallas import tpu as pltpu
from jax.experimental.pallas import tpu_sc as plsc

N_FIELDS = 26
VOCAB = 100000
EMBED_D = 32
BATCH = 16384
TAB_ROWS = N_FIELDS * VOCAB
NUM_CORES = 2
NUM_SUBCORES = 16
NUM_WORKERS = NUM_CORES * NUM_SUBCORES
LANES = 16

GROUP = 4
TAB_GROUPS = TAB_ROWS // GROUP
GROUP_W = GROUP * EMBED_D

TBLK = 1024
NBLK = 2539
TAIL = TAB_ROWS - NBLK * TBLK

ROWS_W = BATCH // NUM_WORKERS
ELEMS_W = ROWS_W * N_FIELDS
GW = 4 * N_FIELDS
NG = ROWS_W // 4
PAIR = 2 * GW
NBUF = 2


def _tree_sum(vals):
    while len(vals) > 1:
        nxt = [vals[i] + vals[i + 1] for i in range(0, len(vals) - 1, 2)]
        if len(vals) % 2:
            nxt.append(vals[-1])
        vals = nxt
    return vals[0]


@functools.partial(
    pl.kernel,
    out_type=jax.ShapeDtypeStruct((TAB_GROUPS, GROUP_W), jnp.float32),
    mesh=plsc.VectorSubcoreMesh(core_axis_name="c", subcore_axis_name="s"),
    compiler_params=pltpu.CompilerParams(needs_layout_passes=False),
    scratch_types=[
        pltpu.VMEM((2, EMBED_D, TBLK), jnp.float32),
        pltpu.VMEM((TBLK // GROUP, GROUP_W), jnp.float32),
        pltpu.VMEM((TAIL // GROUP, GROUP_W), jnp.float32),
        pltpu.SemaphoreType.DMA,
        pltpu.SemaphoreType.DMA,
        pltpu.SemaphoreType.DMA,
    ],
)
def _stage(tab_t_hbm, tail_hbm, out_hbm, in_v, out_t, tail_v,
           sem0, sem1, sem2):
    wid = lax.axis_index("s") * NUM_CORES + lax.axis_index("c")
    sems = (sem0, sem1)
    iota = lax.iota(jnp.int32, LANES)
    iv_row = lax.shift_right_logical(iota, 2)
    iv_col = (iota & 3) * EMBED_D

    @pl.when(wid == 0)
    def _():
        pltpu.sync_copy(tail_hbm, tail_v)
        pltpu.sync_copy(tail_v, out_hbm.at[pl.ds(NBLK * TBLK // GROUP,
                                                 TAIL // GROUP), :])

    nit = (NBLK + NUM_WORKERS - 1) // NUM_WORKERS

    def fetch(i, bslot):
        j = i * NUM_WORKERS + wid

        @pl.when(j < NBLK)
        def _():
            pltpu.async_copy(
                tab_t_hbm.at[:, pl.ds(j * TBLK, TBLK)],
                in_v.at[bslot], sems[bslot])

    def drain(i, bslot):
        j = i * NUM_WORKERS + wid

        @pl.when(j < NBLK)
        def _():
            pltpu.make_async_copy(
                tab_t_hbm.at[:, pl.ds(j * TBLK, TBLK)],
                in_v.at[bslot], sems[bslot]).wait()

    fetch(0, 0)

    def block(i, carry):
        for bslot in range(2):
            j = i * NUM_WORKERS * 2 + bslot * NUM_WORKERS + wid
            ii = 2 * i + bslot
            drain(ii, bslot)
            fetch(ii + 1, 1 - bslot)

            @pl.when(j < NBLK)
            def _():
                def rs_body(rs, c):
                    rows = rs * GROUP + iv_row
                    for d in range(EMBED_D):
                        plsc.store_scatter(
                            out_t, [rows, iv_col + d],
                            in_v[bslot, d, pl.ds(rs * LANES, LANES)])
                    return c
                lax.fori_loop(0, TBLK // LANES, rs_body, 0)
                pltpu.async_copy(
                    out_t, out_hbm.at[pl.ds(j * (TBLK // GROUP),
                                            TBLK // GROUP), :], sem2)
                pltpu.make_async_copy(
                    out_t, out_hbm.at[pl.ds(j * (TBLK // GROUP),
                                            TBLK // GROUP), :], sem2).wait()
        return carry

    lax.fori_loop(0, (nit + 1) // 2, block, 0)


def _gather_body(inp_hbm, tab_hbm, out_hbm, idx_v, gid_v, pat_v, acc_v,
                 buf_v, sem_in, sem0, sem1):
    wid = lax.axis_index("s") * NUM_CORES + lax.axis_index("c")
    sems = (sem0, sem1)

    in_cp = pltpu.async_copy(
        inp_hbm.at[pl.ds(wid * ELEMS_W, ELEMS_W)],
        idx_v.at[pl.ds(0, ELEMS_W)], sem_in)

    iota = lax.iota(jnp.int32, LANES)
    for m in range(PAIR // LANES):
        pat_v[m, :] = ((m * LANES + iota) % N_FIELDS) * VOCAB
    in_cp.wait()

    def adjust(p):
        for m in range(PAIR // LANES):
            sl = pl.ds(p * PAIR + m * LANES, LANES)
            v = idx_v[sl] + pat_v[m, :]
            idx_v[sl] = v
            gid_v[sl] = lax.shift_right_logical(v, 2)

    def start(k, b):
        pltpu.async_copy(
            tab_hbm.at[gid_v.at[pl.ds(k * GW, GW)]], buf_v.at[b], sems[b])

    adjust(0)
    start(0, 0)
    start(1, 1)

    def ring(g, carry):
        for b in range(NBUF):
            k = NBUF * g + b
            pltpu.make_async_copy(
                tab_hbm.at[gid_v.at[pl.ds(k * GW, GW)]], buf_v.at[b],
                sems[b]).wait()
            if b == 0:
                @pl.when(g + 1 < NG // NBUF)
                def _():
                    adjust(g + 1)
            vs = [idx_v[pl.ds(k * GW + j * LANES, LANES)]
                  for j in range(7)]
            for br in range(4):
                arow = 4 * k + br
                cols = [(vs[(br * N_FIELDS + f) // LANES]
                         [(br * N_FIELDS + f) % LANES] & 3) * EMBED_D
                        for f in range(N_FIELDS)]
                for h in range(EMBED_D // LANES):
                    acc_v[pl.ds(arow * EMBED_D + h * LANES, LANES)] = (
                        _tree_sum(
                            [buf_v[b, br * N_FIELDS + f,
                                   pl.ds(cols[f] + h * LANES, LANES)]
                             for f in range(N_FIELDS)]))

            @pl.when(k + NBUF < NG)
            def _():
                start(k + NBUF, b)
        return carry

    lax.fori_loop(0, NG // NBUF, ring, 0)
    pltpu.sync_copy(acc_v, out_hbm.at[pl.ds(wid * ROWS_W * EMBED_D,
                                            ROWS_W * EMBED_D)])


@functools.partial(
    pl.kernel,
    out_type=jax.ShapeDtypeStruct((BATCH * EMBED_D,), jnp.float32),
    mesh=plsc.VectorSubcoreMesh(core_axis_name="c", subcore_axis_name="s"),
    scratch_types=[
        pltpu.VMEM((ELEMS_W + LANES,), jnp.int32),
        pltpu.VMEM((ELEMS_W,), jnp.int32),
        pltpu.VMEM((PAIR // LANES, LANES), jnp.int32),
        pltpu.VMEM((ROWS_W * EMBED_D,), jnp.float32),
        pltpu.VMEM((NBUF, GW, GROUP_W), jnp.float32),
        pltpu.SemaphoreType.DMA,
        pltpu.SemaphoreType.DMA,
        pltpu.SemaphoreType.DMA,
    ],
)
def _attr_embed(inp_hbm, tab_hbm, out_hbm, idx_v, gid_v, pat_v, acc_v,
                buf_v, sem_in, sem0, sem1):
    _gather_body(inp_hbm, tab_hbm, out_hbm, idx_v, gid_v, pat_v, acc_v,
                 buf_v, sem_in, sem0, sem1)


def kernel(inputs, table):
    assert inputs.shape == (BATCH, N_FIELDS) and inputs.dtype == jnp.int32
    tab_t = table.T
    tail = table[NBLK * TBLK:, :].reshape(TAIL // GROUP, GROUP_W)
    staged = _stage(tab_t, tail)
    out = _attr_embed(inputs.reshape(-1), staged)
    return out.reshape(BATCH, EMBED_D)


# speedup vs baseline: 1.6163x; 1.4339x over previous
import functools

import jax
import jax.numpy as jnp
from jax import lax
from jax.experimental import pallas as pl
from jax.experimental.pallas import tpu as pltpu
from jax.experimental.pallas import tpu_sc as plsc

N_FIELDS = 26
VOCAB = 100000
EMBED_D = 32
BATCH = 16384
TAB_ROWS = N_FIELDS * VOCAB
NUM_CORES = 2
NUM_SUBCORES = 16
NUM_WORKERS = NUM_CORES * NUM_SUBCORES
LANES = 16

GROUP = 4
TAB_GROUPS = TAB_ROWS // GROUP
GROUP_W = GROUP * EMBED_D

TBLK = 1024
NBLK = 2539
TAIL = TAB_ROWS - NBLK * TBLK

ROWS_W = BATCH // NUM_WORKERS
ELEMS_W = ROWS_W * N_FIELDS
GW = 4 * N_FIELDS
NG = ROWS_W // 4
PAIR = 2 * GW
NBUF = 2


def _tree_sum(vals):
    while len(vals) > 1:
        nxt = [vals[i] + vals[i + 1] for i in range(0, len(vals) - 1, 2)]
        if len(vals) % 2:
            nxt.append(vals[-1])
        vals = nxt
    return vals[0]


TCBLK = 2048
TC_GRID = -(-TAB_ROWS // TCBLK)


def _tc_stage_body(in_ref, out_ref):
    x = in_ref[...]
    out_ref[:, 0:EMBED_D] = x.T


_tc_stage = pl.pallas_call(
    _tc_stage_body,
    grid=(TC_GRID,),
    in_specs=[pl.BlockSpec((EMBED_D, TCBLK), lambda i: (0, i))],
    out_specs=pl.BlockSpec((TCBLK, GROUP_W), lambda i: (i, 0)),
    out_shape=jax.ShapeDtypeStruct((TAB_ROWS, GROUP_W), jnp.float32),
)


def _gather_body(inp_hbm, tab_hbm, out_hbm, idx_v, pat_v, acc_v,
                 buf_v, sem_in, sem0, sem1):
    wid = lax.axis_index("s") * NUM_CORES + lax.axis_index("c")
    sems = (sem0, sem1)

    in_cp = pltpu.async_copy(
        inp_hbm.at[pl.ds(wid * ELEMS_W, ELEMS_W)],
        idx_v.at[pl.ds(0, ELEMS_W)], sem_in)

    iota = lax.iota(jnp.int32, LANES)
    for m in range(PAIR // LANES):
        pat_v[m, :] = ((m * LANES + iota) % N_FIELDS) * VOCAB
    in_cp.wait()

    def adjust(p):
        for m in range(PAIR // LANES):
            sl = pl.ds(p * PAIR + m * LANES, LANES)
            idx_v[sl] = idx_v[sl] + pat_v[m, :]

    def start(k, b):
        pltpu.async_copy(
            tab_hbm.at[idx_v.at[pl.ds(k * GW, GW)]], buf_v.at[b], sems[b])

    adjust(0)
    start(0, 0)
    start(1, 1)

    def ring(g, carry):
        for b in range(NBUF):
            k = NBUF * g + b
            pltpu.make_async_copy(
                tab_hbm.at[idx_v.at[pl.ds(k * GW, GW)]], buf_v.at[b],
                sems[b]).wait()
            if b == 0:
                @pl.when(g + 1 < NG // NBUF)
                def _():
                    adjust(g + 1)
            for br in range(4):
                arow = 4 * k + br
                for h in range(EMBED_D // LANES):
                    acc_v[pl.ds(arow * EMBED_D + h * LANES, LANES)] = (
                        _tree_sum(
                            [buf_v[b, br * N_FIELDS + f,
                                   pl.ds(h * LANES, LANES)]
                             for f in range(N_FIELDS)]))

            @pl.when(k + NBUF < NG)
            def _():
                start(k + NBUF, b)
        return carry

    lax.fori_loop(0, NG // NBUF, ring, 0)
    pltpu.sync_copy(acc_v, out_hbm.at[pl.ds(wid * ROWS_W * EMBED_D,
                                            ROWS_W * EMBED_D)])


@functools.partial(
    pl.kernel,
    out_type=jax.ShapeDtypeStruct((BATCH * EMBED_D,), jnp.float32),
    mesh=plsc.VectorSubcoreMesh(core_axis_name="c", subcore_axis_name="s"),
    scratch_types=[
        pltpu.VMEM((ELEMS_W,), jnp.int32),
        pltpu.VMEM((PAIR // LANES, LANES), jnp.int32),
        pltpu.VMEM((ROWS_W * EMBED_D,), jnp.float32),
        pltpu.VMEM((NBUF, GW, GROUP_W), jnp.float32),
        pltpu.SemaphoreType.DMA,
        pltpu.SemaphoreType.DMA,
        pltpu.SemaphoreType.DMA,
    ],
)
def _attr_embed(inp_hbm, tab_hbm, out_hbm, idx_v, pat_v, acc_v,
                buf_v, sem_in, sem0, sem1):
    _gather_body(inp_hbm, tab_hbm, out_hbm, idx_v, pat_v, acc_v,
                 buf_v, sem_in, sem0, sem1)


def kernel(inputs, table):
    assert inputs.shape == (BATCH, N_FIELDS) and inputs.dtype == jnp.int32
    tab_t = table.T
    staged = _tc_stage(tab_t)
    out = _attr_embed(inputs.reshape(-1), staged)
    return out.reshape(BATCH, EMBED_D)


# speedup vs baseline: 2.6856x; 1.6615x over previous
import functools

import jax
import jax.numpy as jnp
from jax import lax
from jax.experimental import pallas as pl
from jax.experimental.pallas import tpu as pltpu
from jax.experimental.pallas import tpu_sc as plsc

N_FIELDS = 26
VOCAB = 100000
EMBED_D = 32
BATCH = 16384
TAB_ROWS = N_FIELDS * VOCAB
NUM_CORES = 2
NUM_SUBCORES = 16
NUM_WORKERS = NUM_CORES * NUM_SUBCORES
LANES = 16

GROUP = 4
TAB_GROUPS = TAB_ROWS // GROUP
GROUP_W = GROUP * EMBED_D

TBLK = 1024
NBLK = 2539
TAIL = TAB_ROWS - NBLK * TBLK

ROWS_W = BATCH // NUM_WORKERS
ELEMS_W = ROWS_W * N_FIELDS
GW = 4 * N_FIELDS
NG = ROWS_W // 4
PAIR = 2 * GW
NBUF = 2


def _tree_sum(vals):
    while len(vals) > 1:
        nxt = [vals[i] + vals[i + 1] for i in range(0, len(vals) - 1, 2)]
        if len(vals) % 2:
            nxt.append(vals[-1])
        vals = nxt
    return vals[0]


TCBLK = 8192
TC_GRID = -(-TAB_ROWS // TCBLK)


def _tc_stage_body(in_ref, out_ref):
    x = in_ref[...]
    out_ref[:, 0:EMBED_D] = x.T


_tc_stage = pl.pallas_call(
    _tc_stage_body,
    grid=(TC_GRID,),
    in_specs=[pl.BlockSpec((EMBED_D, TCBLK), lambda i: (0, i))],
    out_specs=pl.BlockSpec((TCBLK, GROUP_W), lambda i: (i, 0)),
    out_shape=jax.ShapeDtypeStruct((TAB_ROWS, GROUP_W), jnp.float32),
)


def _gather_body(inp_hbm, tab_hbm, out_hbm, idx_v, pat_v, acc_v,
                 buf_v, sem_in, sem0, sem1):
    wid = lax.axis_index("s") * NUM_CORES + lax.axis_index("c")
    sems = (sem0, sem1)

    in_cp = pltpu.async_copy(
        inp_hbm.at[pl.ds(wid * ELEMS_W, ELEMS_W)],
        idx_v.at[pl.ds(0, ELEMS_W)], sem_in)

    iota = lax.iota(jnp.int32, LANES)
    for m in range(PAIR // LANES):
        pat_v[m, :] = ((m * LANES + iota) % N_FIELDS) * VOCAB
    in_cp.wait()

    def adjust(p):
        for m in range(PAIR // LANES):
            sl = pl.ds(p * PAIR + m * LANES, LANES)
            idx_v[sl] = idx_v[sl] + pat_v[m, :]

    def start(k, b):
        pltpu.async_copy(
            tab_hbm.at[idx_v.at[pl.ds(k * GW, GW)]], buf_v.at[b], sems[b])

    adjust(0)
    start(0, 0)
    start(1, 1)

    def ring(g, carry):
        for b in range(NBUF):
            k = NBUF * g + b
            pltpu.make_async_copy(
                tab_hbm.at[idx_v.at[pl.ds(k * GW, GW)]], buf_v.at[b],
                sems[b]).wait()
            if b == 0:
                @pl.when(g + 1 < NG // NBUF)
                def _():
                    adjust(g + 1)
            for br in range(4):
                arow = 4 * k + br
                for h in range(EMBED_D // LANES):
                    acc_v[pl.ds(arow * EMBED_D + h * LANES, LANES)] = (
                        _tree_sum(
                            [buf_v[b, br * N_FIELDS + f,
                                   pl.ds(h * LANES, LANES)]
                             for f in range(N_FIELDS)]))

            @pl.when(k + NBUF < NG)
            def _():
                start(k + NBUF, b)
        return carry

    lax.fori_loop(0, NG // NBUF, ring, 0)
    pltpu.sync_copy(acc_v, out_hbm.at[pl.ds(wid * ROWS_W * EMBED_D,
                                            ROWS_W * EMBED_D)])


@functools.partial(
    pl.kernel,
    out_type=jax.ShapeDtypeStruct((BATCH * EMBED_D,), jnp.float32),
    mesh=plsc.VectorSubcoreMesh(core_axis_name="c", subcore_axis_name="s"),
    compiler_params=pltpu.CompilerParams(needs_layout_passes=False),
    scratch_types=[
        pltpu.VMEM((ELEMS_W,), jnp.int32),
        pltpu.VMEM((PAIR // LANES, LANES), jnp.int32),
        pltpu.VMEM((ROWS_W * EMBED_D,), jnp.float32),
        pltpu.VMEM((NBUF, GW, GROUP_W), jnp.float32),
        pltpu.SemaphoreType.DMA,
        pltpu.SemaphoreType.DMA,
        pltpu.SemaphoreType.DMA,
    ],
)
def _attr_embed(inp_hbm, tab_hbm, out_hbm, idx_v, pat_v, acc_v,
                buf_v, sem_in, sem0, sem1):
    _gather_body(inp_hbm, tab_hbm, out_hbm, idx_v, pat_v, acc_v,
                 buf_v, sem_in, sem0, sem1)


def kernel(inputs, table):
    assert inputs.shape == (BATCH, N_FIELDS) and inputs.dtype == jnp.int32
    tab_t = table.T
    staged = _tc_stage(tab_t)
    out = _attr_embed(inputs.reshape(-1), staged)
    return out.reshape(BATCH, EMBED_D)


# speedup vs baseline: 3.0249x; 1.1263x over previous
import functools

import jax
import jax.numpy as jnp
from jax import lax
from jax.experimental import pallas as pl
from jax.experimental.pallas import tpu as pltpu
from jax.experimental.pallas import tpu_sc as plsc

N_FIELDS = 26
VOCAB = 100000
EMBED_D = 32
BATCH = 16384
TAB_ROWS = N_FIELDS * VOCAB
NUM_CORES = 2
NUM_SUBCORES = 16
NUM_WORKERS = NUM_CORES * NUM_SUBCORES
LANES = 16

GROUP = 4
TAB_GROUPS = TAB_ROWS // GROUP
GROUP_W = GROUP * EMBED_D

TBLK = 1024
NBLK = 2539
TAIL = TAB_ROWS - NBLK * TBLK

ROWS_W = BATCH // NUM_WORKERS
ELEMS_W = ROWS_W * N_FIELDS
GW = 4 * N_FIELDS
NG = ROWS_W // 4
PAIR = 2 * GW
NBUF = 2


def _tree_sum(vals):
    while len(vals) > 1:
        nxt = [vals[i] + vals[i + 1] for i in range(0, len(vals) - 1, 2)]
        if len(vals) % 2:
            nxt.append(vals[-1])
        vals = nxt
    return vals[0]


TCBLK = 16384
TC_GRID = -(-TAB_ROWS // TCBLK)


def _tc_stage_body(in_ref, out_ref):
    x = in_ref[...]
    out_ref[:, 0:EMBED_D] = x.T


_tc_stage = pl.pallas_call(
    _tc_stage_body,
    grid=(TC_GRID,),
    in_specs=[pl.BlockSpec((EMBED_D, TCBLK), lambda i: (0, i))],
    out_specs=pl.BlockSpec((TCBLK, GROUP_W), lambda i: (i, 0)),
    out_shape=jax.ShapeDtypeStruct((TAB_ROWS, GROUP_W), jnp.float32),
)


def _gather_body(inp_hbm, tab_hbm, out_hbm, idx_v, pat_v, acc_v,
                 buf_v, sem_in, sem0, sem1):
    wid = lax.axis_index("s") * NUM_CORES + lax.axis_index("c")
    sems = (sem0, sem1)

    in_cp = pltpu.async_copy(
        inp_hbm.at[pl.ds(wid * ELEMS_W, ELEMS_W)],
        idx_v.at[pl.ds(0, ELEMS_W)], sem_in)

    iota = lax.iota(jnp.int32, LANES)
    for m in range(PAIR // LANES):
        pat_v[m, :] = ((m * LANES + iota) % N_FIELDS) * VOCAB
    in_cp.wait()

    def adjust(p):
        for m in range(PAIR // LANES):
            sl = pl.ds(p * PAIR + m * LANES, LANES)
            idx_v[sl] = idx_v[sl] + pat_v[m, :]

    def start(k, b):
        pltpu.async_copy(
            tab_hbm.at[idx_v.at[pl.ds(k * GW, GW)]], buf_v.at[b], sems[b])

    adjust(0)
    start(0, 0)
    start(1, 1)

    def ring(g, carry):
        for b in range(NBUF):
            k = NBUF * g + b
            pltpu.make_async_copy(
                tab_hbm.at[idx_v.at[pl.ds(k * GW, GW)]], buf_v.at[b],
                sems[b]).wait()
            if b == 0:
                @pl.when(g + 1 < NG // NBUF)
                def _():
                    adjust(g + 1)
            for br in range(4):
                arow = 4 * k + br
                for h in range(EMBED_D // LANES):
                    acc_v[pl.ds(arow * EMBED_D + h * LANES, LANES)] = (
                        _tree_sum(
                            [buf_v[b, br * N_FIELDS + f,
                                   pl.ds(h * LANES, LANES)]
                             for f in range(N_FIELDS)]))

            @pl.when(k + NBUF < NG)
            def _():
                start(k + NBUF, b)
        return carry

    lax.fori_loop(0, NG // NBUF, ring, 0)
    pltpu.sync_copy(acc_v, out_hbm.at[pl.ds(wid * ROWS_W * EMBED_D,
                                            ROWS_W * EMBED_D)])


@functools.partial(
    pl.kernel,
    out_type=jax.ShapeDtypeStruct((BATCH * EMBED_D,), jnp.float32),
    mesh=plsc.VectorSubcoreMesh(core_axis_name="c", subcore_axis_name="s"),
    compiler_params=pltpu.CompilerParams(needs_layout_passes=False),
    scratch_types=[
        pltpu.VMEM((ELEMS_W,), jnp.int32),
        pltpu.VMEM((PAIR // LANES, LANES), jnp.int32),
        pltpu.VMEM((ROWS_W * EMBED_D,), jnp.float32),
        pltpu.VMEM((NBUF, GW, GROUP_W), jnp.float32),
        pltpu.SemaphoreType.DMA,
        pltpu.SemaphoreType.DMA,
        pltpu.SemaphoreType.DMA,
    ],
)
def _attr_embed(inp_hbm, tab_hbm, out_hbm, idx_v, pat_v, acc_v,
                buf_v, sem_in, sem0, sem1):
    _gather_body(inp_hbm, tab_hbm, out_hbm, idx_v, pat_v, acc_v,
                 buf_v, sem_in, sem0, sem1)


def kernel(inputs, table):
    assert inputs.shape == (BATCH, N_FIELDS) and inputs.dtype == jnp.int32
    tab_t = table.T
    staged = _tc_stage(tab_t)
    out = _attr_embed(inputs.reshape(-1), staged)
    return out.reshape(BATCH, EMBED_D)


# speedup vs baseline: 3.0957x; 1.0234x over previous
import functools

import jax
import jax.numpy as jnp
from jax import lax
from jax.experimental import pallas as pl
from jax.experimental.pallas import tpu as pltpu
from jax.experimental.pallas import tpu_sc as plsc

N_FIELDS = 26
VOCAB = 100000
EMBED_D = 32
BATCH = 16384
TAB_ROWS = N_FIELDS * VOCAB
NUM_CORES = 2
NUM_SUBCORES = 16
NUM_WORKERS = NUM_CORES * NUM_SUBCORES
LANES = 16

GROUP = 4
TAB_GROUPS = TAB_ROWS // GROUP
GROUP_W = GROUP * EMBED_D

TBLK = 1024
NBLK = 2539
TAIL = TAB_ROWS - NBLK * TBLK

ROWS_W = BATCH // NUM_WORKERS
ELEMS_W = ROWS_W * N_FIELDS
GW = 4 * N_FIELDS
NG = ROWS_W // 4
PAIR = 2 * GW
NBUF = 2


def _tree_sum(vals):
    while len(vals) > 1:
        nxt = [vals[i] + vals[i + 1] for i in range(0, len(vals) - 1, 2)]
        if len(vals) % 2:
            nxt.append(vals[-1])
        vals = nxt
    return vals[0]


TCBLK = 32768
TC_GRID = -(-TAB_ROWS // TCBLK)


def _tc_stage_body(in_ref, out_ref):
    x = in_ref[...]
    out_ref[:, 0:EMBED_D] = x.T


_tc_stage = pl.pallas_call(
    _tc_stage_body,
    grid=(TC_GRID,),
    in_specs=[pl.BlockSpec((EMBED_D, TCBLK), lambda i: (0, i))],
    out_specs=pl.BlockSpec((TCBLK, GROUP_W), lambda i: (i, 0)),
    out_shape=jax.ShapeDtypeStruct((TAB_ROWS, GROUP_W), jnp.float32),
)


def _gather_body(inp_hbm, tab_hbm, out_hbm, idx_v, pat_v, acc_v,
                 buf_v, sem_in, sem0, sem1):
    wid = lax.axis_index("s") * NUM_CORES + lax.axis_index("c")
    sems = (sem0, sem1)

    in_cp = pltpu.async_copy(
        inp_hbm.at[pl.ds(wid * ELEMS_W, ELEMS_W)],
        idx_v.at[pl.ds(0, ELEMS_W)], sem_in)

    iota = lax.iota(jnp.int32, LANES)
    for m in range(PAIR // LANES):
        pat_v[m, :] = ((m * LANES + iota) % N_FIELDS) * VOCAB
    in_cp.wait()

    def adjust(p):
        for m in range(PAIR // LANES):
            sl = pl.ds(p * PAIR + m * LANES, LANES)
            idx_v[sl] = idx_v[sl] + pat_v[m, :]

    def start(k, b):
        pltpu.async_copy(
            tab_hbm.at[idx_v.at[pl.ds(k * GW, GW)]], buf_v.at[b], sems[b])

    adjust(0)
    start(0, 0)
    start(1, 1)

    def ring(g, carry):
        for b in range(NBUF):
            k = NBUF * g + b
            pltpu.make_async_copy(
                tab_hbm.at[idx_v.at[pl.ds(k * GW, GW)]], buf_v.at[b],
                sems[b]).wait()
            if b == 0:
                @pl.when(g + 1 < NG // NBUF)
                def _():
                    adjust(g + 1)
            for br in range(4):
                arow = 4 * k + br
                for h in range(EMBED_D // LANES):
                    acc_v[pl.ds(arow * EMBED_D + h * LANES, LANES)] = (
                        _tree_sum(
                            [buf_v[b, br * N_FIELDS + f,
                                   pl.ds(h * LANES, LANES)]
                             for f in range(N_FIELDS)]))

            @pl.when(k + NBUF < NG)
            def _():
                start(k + NBUF, b)
        return carry

    lax.fori_loop(0, NG // NBUF, ring, 0)
    pltpu.sync_copy(acc_v, out_hbm.at[pl.ds(wid * ROWS_W * EMBED_D,
                                            ROWS_W * EMBED_D)])


@functools.partial(
    pl.kernel,
    out_type=jax.ShapeDtypeStruct((BATCH * EMBED_D,), jnp.float32),
    mesh=plsc.VectorSubcoreMesh(core_axis_name="c", subcore_axis_name="s"),
    compiler_params=pltpu.CompilerParams(needs_layout_passes=False),
    scratch_types=[
        pltpu.VMEM((ELEMS_W,), jnp.int32),
        pltpu.VMEM((PAIR // LANES, LANES), jnp.int32),
        pltpu.VMEM((ROWS_W * EMBED_D,), jnp.float32),
        pltpu.VMEM((NBUF, GW, GROUP_W), jnp.float32),
        pltpu.SemaphoreType.DMA,
        pltpu.SemaphoreType.DMA,
        pltpu.SemaphoreType.DMA,
    ],
)
def _attr_embed(inp_hbm, tab_hbm, out_hbm, idx_v, pat_v, acc_v,
                buf_v, sem_in, sem0, sem1):
    _gather_body(inp_hbm, tab_hbm, out_hbm, idx_v, pat_v, acc_v,
                 buf_v, sem_in, sem0, sem1)


def kernel(inputs, table):
    assert inputs.shape == (BATCH, N_FIELDS) and inputs.dtype == jnp.int32
    tab_t = table.T
    staged = _tc_stage(tab_t)
    out = _attr_embed(inputs.reshape(-1), staged)
    return out.reshape(BATCH, EMBED_D)
